# baseline (speedup 1.0000x reference)
import numpy as np
import jax
import jax.numpy as jnp
from jax import lax
from jax.experimental import pallas as pl
from jax.experimental.pallas import tpu as pltpu
from jax.experimental.pallas import tpu_sc as plsc

DIM = 128

_P2 = (1.0, -0.6931470632553101, 0.24022436141967773, -0.05549103766679764,
       0.009580060094594955, -0.0012757170759141445, 0.00010938758234521374)
_LOG2E = 1.4426950408889634
_POW2 = np.ldexp(np.float32(1.0), -np.arange(128)).astype(np.float32)


def _skipgram_scs(emb_hbm, iw_hbm, ow_hbm, sign_hbm, pow2_hbm, out_hbm,
                  iw_s, ow_s, sign_s, pow2_s, row0_s, row1_s, out_s,
                  sem0, sem1, sem2, sem3):
    c0 = pltpu.make_async_copy(iw_hbm, iw_s, sem0)
    c1 = pltpu.make_async_copy(ow_hbm, ow_s, sem1)
    c2 = pltpu.make_async_copy(sign_hbm, sign_s, sem2)
    c3 = pltpu.make_async_copy(pow2_hbm, pow2_s, sem3)
    c0.start()
    c1.start()
    c2.start()
    c3.start()
    c0.wait()
    g0 = pltpu.make_async_copy(emb_hbm.at[pl.ds(iw_s[0], 1)], row0_s, sem0)
    g0.start()
    c1.wait()
    g1 = pltpu.make_async_copy(emb_hbm.at[pl.ds(ow_s[0], 1)], row1_s, sem1)
    g1.start()
    g0.wait()
    g1.wait()
    accs = [row0_s[0, a] * row1_s[0, a] for a in range(4)]
    for j in range(4, DIM, 4):
        for a in range(4):
            accs[a] = accs[a] + row0_s[0, j + a] * row1_s[0, j + a]
    acc = (accs[0] + accs[1]) + (accs[2] + accs[3])
    c2.wait()
    c3.wait()
    x = acc * sign_s[0]
    z = jnp.minimum(jnp.maximum(x, -x), jnp.float32(80.0))
    y = z * jnp.float32(_LOG2E)
    n = jnp.minimum(y.astype(jnp.int32), jnp.int32(120))
    f = y - n.astype(jnp.float32)
    p2 = jnp.float32(_P2[6])
    for k in (5, 4, 3, 2, 1, 0):
        p2 = p2 * f + jnp.float32(_P2[k])
    u = p2 * pow2_s[n]
    d = u + jnp.float32(2.0)
    r = jnp.float32(12.0 / 17.0) - jnp.float32(2.0 / 17.0) * d
    for _ in range(3):
        r = r * (jnp.float32(2.0) - d * r)
    t = u * r
    t2 = t * t
    log1p_u = jnp.float32(2.0) * t * (
        jnp.float32(1.0) + t2 * (jnp.float32(1.0 / 3.0) + t2 * (
            jnp.float32(1.0 / 5.0) + t2 * (jnp.float32(1.0 / 7.0)
                                           + t2 * jnp.float32(1.0 / 9.0)))))
    out_s[0] = jnp.minimum(x, jnp.float32(0.0)) - log1p_u
    pltpu.sync_copy(out_s, out_hbm)


def kernel(input_word, output_word, sign, emb):
    out = pl.kernel(
        _skipgram_scs,
        out_type=jax.ShapeDtypeStruct((1,), jnp.float32),
        mesh=plsc.ScalarSubcoreMesh(axis_name="c", num_cores=1),
        compiler_params=pltpu.CompilerParams(needs_layout_passes=False),
        scratch_types=[
            pltpu.SMEM((1,), jnp.int32),
            pltpu.SMEM((1,), jnp.int32),
            pltpu.SMEM((1,), jnp.float32),
            pltpu.SMEM((128,), jnp.float32),
            pltpu.SMEM((1, DIM), jnp.float32),
            pltpu.SMEM((1, DIM), jnp.float32),
            pltpu.SMEM((1,), jnp.float32),
            pltpu.SemaphoreType.DMA,
            pltpu.SemaphoreType.DMA,
            pltpu.SemaphoreType.DMA,
            pltpu.SemaphoreType.DMA,
        ],
    )(emb, input_word.astype(jnp.int32), output_word.astype(jnp.int32),
      sign.reshape(1), jnp.asarray(_POW2))
    return out.reshape(())


# speedup vs baseline: 1.0057x; 1.0057x over previous
import numpy as np
import jax
import jax.numpy as jnp
from jax import lax
from jax.experimental import pallas as pl
from jax.experimental.pallas import tpu as pltpu
from jax.experimental.pallas import tpu_sc as plsc

DIM = 128

_P2 = (1.0, -0.6931470632553101, 0.24022436141967773, -0.05549103766679764,
       0.009580060094594955, -0.0012757170759141445, 0.00010938758234521374)
_LOG2E = 1.4426950408889634
_POW2 = np.ldexp(np.float32(1.0), -np.arange(128)).astype(np.float32)


def _skipgram_scs(emb_hbm, iw_hbm, ow_hbm, sign_hbm, pow2_hbm, out_hbm,
                  iw_s, ow_s, sign_s, pow2_s, row0_s, row1_s, out_s,
                  sem0, sem1, sem2, sem3):
    c0 = pltpu.make_async_copy(iw_hbm, iw_s, sem0)
    c1 = pltpu.make_async_copy(ow_hbm, ow_s, sem1)
    c2 = pltpu.make_async_copy(sign_hbm, sign_s, sem2)
    c3 = pltpu.make_async_copy(pow2_hbm, pow2_s, sem3)
    c0.start()
    c1.start()
    c2.start()
    c3.start()
    c0.wait()
    g0 = pltpu.make_async_copy(emb_hbm.at[pl.ds(iw_s[0], 1)], row0_s, sem0)
    g0.start()
    c1.wait()
    g1 = pltpu.make_async_copy(emb_hbm.at[pl.ds(ow_s[0], 1)], row1_s, sem1)
    g1.start()
    g0.wait()
    g1.wait()
    def dot_body(i, accs):
        base = i * 8
        return tuple(
            accs[a] + row0_s[0, base + 2 * a] * row1_s[0, base + 2 * a]
            + row0_s[0, base + 2 * a + 1] * row1_s[0, base + 2 * a + 1]
            for a in range(4)
        )
    accs = lax.fori_loop(
        0, DIM // 8, dot_body,
        (jnp.float32(0.0), jnp.float32(0.0), jnp.float32(0.0), jnp.float32(0.0)))
    acc = (accs[0] + accs[1]) + (accs[2] + accs[3])
    c2.wait()
    c3.wait()
    x = acc * sign_s[0]
    z = jnp.minimum(jnp.maximum(x, -x), jnp.float32(80.0))
    y = z * jnp.float32(_LOG2E)
    n = jnp.minimum(y.astype(jnp.int32), jnp.int32(120))
    f = y - n.astype(jnp.float32)
    p2 = jnp.float32(_P2[6])
    for k in (5, 4, 3, 2, 1, 0):
        p2 = p2 * f + jnp.float32(_P2[k])
    u = p2 * pow2_s[n]
    d = u + jnp.float32(2.0)
    r = jnp.float32(12.0 / 17.0) - jnp.float32(2.0 / 17.0) * d
    for _ in range(3):
        r = r * (jnp.float32(2.0) - d * r)
    t = u * r
    t2 = t * t
    log1p_u = jnp.float32(2.0) * t * (
        jnp.float32(1.0) + t2 * (jnp.float32(1.0 / 3.0) + t2 * (
            jnp.float32(1.0 / 5.0) + t2 * (jnp.float32(1.0 / 7.0)
                                           + t2 * jnp.float32(1.0 / 9.0)))))
    out_s[0] = jnp.minimum(x, jnp.float32(0.0)) - log1p_u
    pltpu.sync_copy(out_s, out_hbm)


def kernel(input_word, output_word, sign, emb):
    out = pl.kernel(
        _skipgram_scs,
        out_type=jax.ShapeDtypeStruct((1,), jnp.float32),
        mesh=plsc.ScalarSubcoreMesh(axis_name="c", num_cores=1),
        compiler_params=pltpu.CompilerParams(needs_layout_passes=False),
        scratch_types=[
            pltpu.SMEM((1,), jnp.int32),
            pltpu.SMEM((1,), jnp.int32),
            pltpu.SMEM((1,), jnp.float32),
            pltpu.SMEM((128,), jnp.float32),
            pltpu.SMEM((1, DIM), jnp.float32),
            pltpu.SMEM((1, DIM), jnp.float32),
            pltpu.SMEM((1,), jnp.float32),
            pltpu.SemaphoreType.DMA,
            pltpu.SemaphoreType.DMA,
            pltpu.SemaphoreType.DMA,
            pltpu.SemaphoreType.DMA,
        ],
    )(emb, input_word.astype(jnp.int32), output_word.astype(jnp.int32),
      sign.reshape(1), jnp.asarray(_POW2))
    return out.reshape(())
